# baseline (speedup 1.0000x reference)
import functools

import jax
import jax.numpy as jnp
from jax import lax
from jax.experimental import pallas as pl
from jax.experimental.pallas import tpu as pltpu
from jax.experimental.pallas import tpu_sc as plsc

BATCH = 16384
USER_DIM = 32
ITEM_DIM = 64
OUT_DIM = USER_DIM + ITEM_DIM

_NC = 2
_NS = 16
_NW = _NC * _NS
_BPW = BATCH // _NW


def _make_kernel():
    mesh = plsc.VectorSubcoreMesh(core_axis_name="c", subcore_axis_name="s")

    @functools.partial(
        pl.kernel,
        mesh=mesh,
        out_type=jax.ShapeDtypeStruct((BATCH, OUT_DIM), jnp.float32),
        compiler_params=pltpu.CompilerParams(use_tc_tiling_on_sc=False),
        scratch_types=[
            pltpu.VMEM((_BPW,), jnp.int32),
            pltpu.VMEM((_BPW,), jnp.int32),
            pltpu.VMEM((_BPW, USER_DIM), jnp.float32),
            pltpu.VMEM((_BPW, ITEM_DIM), jnp.float32),
            pltpu.SemaphoreType.DMA,
            pltpu.SemaphoreType.DMA,
        ],
    )
    def gather_concat(user_idx_hbm, item_idx_hbm, user_tbl_hbm, item_tbl_hbm,
                      out_hbm, uidx_v, iidx_v, urows_v, irows_v, usem, isem):
        wid = lax.axis_index("s") * _NC + lax.axis_index("c")
        base = wid * _BPW
        pltpu.sync_copy(user_idx_hbm.at[pl.ds(base, _BPW)], uidx_v)
        pltpu.sync_copy(item_idx_hbm.at[pl.ds(base, _BPW)], iidx_v)
        ucp = pltpu.async_copy(user_tbl_hbm.at[uidx_v], urows_v, usem)
        icp = pltpu.async_copy(item_tbl_hbm.at[iidx_v], irows_v, isem)
        ucp.wait()
        pltpu.sync_copy(urows_v,
                        out_hbm.at[pl.ds(base, _BPW), pl.ds(0, USER_DIM)])
        icp.wait()
        pltpu.sync_copy(irows_v,
                        out_hbm.at[pl.ds(base, _BPW), pl.ds(USER_DIM, ITEM_DIM)])

    return gather_concat


_gather_concat = _make_kernel()


def kernel(user_input, item_input, user_table, item_table):
    return _gather_concat(user_input.astype(jnp.int32),
                          item_input.astype(jnp.int32),
                          user_table, item_table)
